# baseline (speedup 1.0000x reference)
import jax
import jax.numpy as jnp
from jax import lax
from jax.experimental import pallas as pl
from jax.experimental.pallas import tpu as pltpu
from jax.experimental.pallas import tpu_sc as plsc

B = 16384
D = 64
NC = 2
NS = 16
NW = NC * NS
BPW = B // NW
QB = 128


def _emb_body(xu_hbm, xp_hbm, xn_hbm, uid_hbm, iid_hbm, out_hbm,
              iu_v, ip_v, in_v, big_v, cu_v, cp_v, cn_v, sem):
    c = lax.axis_index("c")
    s = lax.axis_index("s")
    wid = s * NC + c
    base = wid * BPW
    pltpu.sync_copy(xu_hbm.at[pl.ds(base, BPW)], iu_v)
    pltpu.sync_copy(xp_hbm.at[pl.ds(base, BPW)], ip_v)
    pltpu.sync_copy(xn_hbm.at[pl.ds(base, BPW)], in_v)

    def quarter(q, carry):
        @plsc.parallel_loop(0, QB // 16, unroll=2)
        def group(g):
            r = q * QB + g * 16
            vu = iu_v[pl.ds(r, 16)]
            vp = ip_v[pl.ds(r, 16)]
            vn = in_v[pl.ds(r, 16)]
            for j in range(16):
                d = 3 * (g * 16 + j)
                pltpu.async_copy(uid_hbm.at[pl.ds(vu[j], 1)],
                                 big_v.at[pl.ds(d, 1)], sem)
                pltpu.async_copy(iid_hbm.at[pl.ds(vp[j], 1)],
                                 big_v.at[pl.ds(d + 1, 1)], sem)
                pltpu.async_copy(iid_hbm.at[pl.ds(vn[j], 1)],
                                 big_v.at[pl.ds(d + 2, 1)], sem)

        pltpu.make_async_copy(uid_hbm.at[pl.ds(0, 3 * QB)], big_v, sem).wait()

        lanes = lax.iota(jnp.int32, 16) * 3
        for k in range(D):
            kcol = jnp.full((16,), k, jnp.int32)
            for g in range(QB // 16):
                rows = lanes + 3 * g * 16
                sl = pl.ds(g * 16, 16)
                cu_v[k, sl] = plsc.load_gather(big_v, [rows, kcol])
                cp_v[k, sl] = plsc.load_gather(big_v, [rows + 1, kcol])
                cn_v[k, sl] = plsc.load_gather(big_v, [rows + 2, kcol])
        b0 = base + q * QB
        pltpu.sync_copy(cu_v, out_hbm.at[0, :, pl.ds(b0, QB)])
        pltpu.sync_copy(cp_v, out_hbm.at[1, :, pl.ds(b0, QB)])
        pltpu.sync_copy(cn_v, out_hbm.at[2, :, pl.ds(b0, QB)])
        return carry

    lax.fori_loop(0, BPW // QB, quarter, 0)


def kernel(x, uid_table, iid_table):
    x = x.astype(jnp.int32)
    xu = x[:, 0]
    xp = x[:, 1]
    xn = x[:, 2]
    mesh = plsc.VectorSubcoreMesh(core_axis_name="c", subcore_axis_name="s")
    k = pl.kernel(
        _emb_body,
        out_type=jax.ShapeDtypeStruct((3, D, B), jnp.float32),
        mesh=mesh,
        compiler_params=pltpu.CompilerParams(
            use_tc_tiling_on_sc=True,
            needs_layout_passes=False,
        ),
        scratch_types=[
            pltpu.VMEM((BPW,), jnp.int32),
            pltpu.VMEM((BPW,), jnp.int32),
            pltpu.VMEM((BPW,), jnp.int32),
            pltpu.VMEM((3 * QB, D), jnp.float32),
            pltpu.VMEM((D, QB), jnp.float32),
            pltpu.VMEM((D, QB), jnp.float32),
            pltpu.VMEM((D, QB), jnp.float32),
            pltpu.SemaphoreType.DMA,
        ],
    )
    out = k(xu, xp, xn, uid_table, iid_table)
    return jnp.transpose(out, (2, 0, 1))


# speedup vs baseline: 1.0339x; 1.0339x over previous
import jax
import jax.numpy as jnp
from jax import lax
from jax.experimental import pallas as pl
from jax.experimental.pallas import tpu as pltpu
from jax.experimental.pallas import tpu_sc as plsc

B = 16384
D = 64
NC = 2
NS = 16
NW = NC * NS
BPW = B // NW
HB = BPW // 2


def _emb_body(xu_hbm, xp_hbm, xn_hbm, uid_hbm, iid_hbm, out_hbm,
              iu_v, ip_v, in_v, big_v, sem):
    c = lax.axis_index("c")
    s = lax.axis_index("s")
    wid = s * NC + c
    base = wid * BPW
    pltpu.sync_copy(xu_hbm.at[pl.ds(base, BPW)], iu_v)
    pltpu.sync_copy(xp_hbm.at[pl.ds(base, BPW)], ip_v)
    pltpu.sync_copy(xn_hbm.at[pl.ds(base, BPW)], in_v)

    def half(h, carry):
        @plsc.parallel_loop(0, HB // 16, unroll=2)
        def group(g):
            r = h * HB + g * 16
            vu = iu_v[pl.ds(r, 16)]
            vp = ip_v[pl.ds(r, 16)]
            vn = in_v[pl.ds(r, 16)]
            for j in range(16):
                d = 3 * (g * 16 + j)
                pltpu.async_copy(uid_hbm.at[pl.ds(vu[j], 1)],
                                 big_v.at[pl.ds(d, 1)], sem)
                pltpu.async_copy(iid_hbm.at[pl.ds(vp[j], 1)],
                                 big_v.at[pl.ds(d + 1, 1)], sem)
                pltpu.async_copy(iid_hbm.at[pl.ds(vn[j], 1)],
                                 big_v.at[pl.ds(d + 2, 1)], sem)

        pltpu.make_async_copy(uid_hbm.at[pl.ds(0, 3 * HB)], big_v, sem).wait()
        pltpu.sync_copy(big_v, out_hbm.at[pl.ds(3 * (base + h * HB), 3 * HB)])
        return carry

    lax.fori_loop(0, 2, half, 0)


def kernel(x, uid_table, iid_table):
    x = x.astype(jnp.int32)
    xu = x[:, 0]
    xp = x[:, 1]
    xn = x[:, 2]
    mesh = plsc.VectorSubcoreMesh(core_axis_name="c", subcore_axis_name="s")
    k = pl.kernel(
        _emb_body,
        out_type=jax.ShapeDtypeStruct((3 * B, D), jnp.float32),
        mesh=mesh,
        compiler_params=pltpu.CompilerParams(use_tc_tiling_on_sc=True),
        scratch_types=[
            pltpu.VMEM((BPW,), jnp.int32),
            pltpu.VMEM((BPW,), jnp.int32),
            pltpu.VMEM((BPW,), jnp.int32),
            pltpu.VMEM((3 * HB, D), jnp.float32),
            pltpu.SemaphoreType.DMA,
        ],
    )
    out = k(xu, xp, xn, uid_table, iid_table)
    return out.reshape(B, 3, D)
